# Initial kernel scaffold; baseline (speedup 1.0000x reference)
import jax
import jax.numpy as jnp
from jax.experimental import pallas as pl


def kernel(x, edge_index, batch, W1, as1, ad1, b1, W2, as2, ad2, b2, W3, as3, ad3, b3, Wn, bn, W0, b0, W4, b4):
    raise NotImplementedError("write your pallas kernel here")


# speedup vs baseline: 1.2524x; 1.2524x over previous
import functools

import jax
import jax.numpy as jnp
from jax.experimental import pallas as pl
from jax.experimental.pallas import tpu as pltpu

N_NODES = 10000
N_EDGES = 320000
N_GRAPHS = 64


def _mm_body(x_ref, w_ref, o_ref):
    o_ref[...] = jnp.dot(x_ref[...], w_ref[...], preferred_element_type=jnp.float32)


def _mm(x, w, blk=1000):
    m, k = x.shape
    _, n = w.shape
    return pl.pallas_call(
        _mm_body,
        grid=(m // blk,),
        in_specs=[
            pl.BlockSpec((blk, k), lambda i: (i, 0)),
            pl.BlockSpec((k, n), lambda i: (0, 0)),
        ],
        out_specs=pl.BlockSpec((blk, n), lambda i: (i, 0)),
        out_shape=jax.ShapeDtypeStruct((m, n), jnp.float32),
    )(x, w)


def _gat_conv(x, src, dst, W, a_src, a_dst, b):
    n = x.shape[0]
    h = _mm(x, W)
    alpha_src = h @ a_src
    alpha_dst = h @ a_dst
    e = jax.nn.leaky_relu(alpha_src[src] + alpha_dst[dst], negative_slope=0.2)
    e_max = jax.ops.segment_max(e, dst, num_segments=n)
    e_exp = jnp.exp(e - e_max[dst])
    denom = jax.ops.segment_sum(e_exp, dst, num_segments=n)
    alpha = e_exp / (denom[dst] + 1e-16)
    out = jax.ops.segment_sum(h[src] * alpha[:, None], dst, num_segments=n)
    return out + b


def kernel(x, edge_index, batch, W1, as1, ad1, b1, W2, as2, ad2, b2, W3, as3, ad3, b3, Wn, bn, W0, b0, W4, b4):
    src, dst = edge_index[0], edge_index[1]
    h = jax.nn.relu(_gat_conv(x, src, dst, W1, as1, ad1, b1))
    h = jax.nn.relu(_gat_conv(h, src, dst, W2, as2, ad2, b2))
    h = jax.nn.relu(_gat_conv(h, src, dst, W3, as3, ad3, b3))
    hp = jax.ops.segment_max(h, batch, num_segments=N_GRAPHS)
    hp = jax.nn.relu(hp @ W0 + b0)
    diff = batch[1:] - batch[:-1]
    root = jnp.nonzero(diff, size=N_GRAPHS - 1)[0]
    root = jnp.concatenate([jnp.zeros((1,), dtype=root.dtype), root + 1])
    news = x[root]
    news = jax.nn.relu(news @ Wn + bn)
    out = jnp.concatenate([hp, news], axis=-1) @ W4 + b4
    return jax.nn.sigmoid(out)


# speedup vs baseline: 6.9821x; 5.5750x over previous
import functools

import jax
import jax.numpy as jnp
from jax import lax
from jax.experimental import pallas as pl
from jax.experimental.pallas import tpu as pltpu
from jax.experimental.pallas import tpu_sc as plsc

NN = 10000
HD = 64
NNP = 10240
NE = 320000
NEP = 327680
D = 128
NG = 64
NC = 2
NS = 16
ROWS_T = 160
ROWS_W = 80
NTAB = NNP // 128
NNA = 10112
NBIN = 80


def _mesh():
    return plsc.VectorSubcoreMesh(core_axis_name="c", subcore_axis_name="s")


def _hi(v):
    return lax.shift_right_logical(v, 7)


def _lo(v):
    return jnp.bitwise_and(v, 127)


def _prep_first_body(x_ref, w_ref, as_ref, ad_ref, hw_ref, asv_ref, adv_ref):
    hw = jnp.dot(x_ref[...], w_ref[...], preferred_element_type=jnp.float32)
    hw_ref[...] = hw
    asv_ref[...] = jnp.sum(hw * as_ref[...], axis=1, keepdims=True)
    adv_ref[...] = jnp.sum(hw * ad_ref[...], axis=1, keepdims=True)


def _prep_first(x, W, a_s, a_d):
    blk = 1000
    return pl.pallas_call(
        _prep_first_body,
        grid=(NN // blk,),
        in_specs=[
            pl.BlockSpec((blk, D), lambda i: (i, 0)),
            pl.BlockSpec((D, D), lambda i: (0, 0)),
            pl.BlockSpec((1, D), lambda i: (0, 0)),
            pl.BlockSpec((1, D), lambda i: (0, 0)),
        ],
        out_specs=[
            pl.BlockSpec((blk, D), lambda i: (i, 0)),
            pl.BlockSpec((blk, 1), lambda i: (i, 0)),
            pl.BlockSpec((blk, 1), lambda i: (i, 0)),
        ],
        out_shape=[
            jax.ShapeDtypeStruct((NN, D), jnp.float32),
            jax.ShapeDtypeStruct((NN, 1), jnp.float32),
            jax.ShapeDtypeStruct((NN, 1), jnp.float32),
        ],
    )(x, W, a_s, a_d)


def _halves_to_h(p_ref, d_ref, b_ref):
    q = p_ref[0] + p_ref[1]
    p = jnp.concatenate([q[0], q[1]], axis=-1)
    p = p / (d_ref[...] + 1e-16) + b_ref[...]
    return jnp.maximum(p, 0.0)


def _prep_mid_body(p_ref, d_ref, b_ref, w_ref, as_ref, ad_ref,
                   hw_ref, asv_ref, adv_ref):
    h = _halves_to_h(p_ref, d_ref, b_ref)
    hw = jnp.dot(h, w_ref[...], preferred_element_type=jnp.float32)
    hw_ref[...] = hw
    asv_ref[...] = jnp.sum(hw * as_ref[...], axis=1, keepdims=True)
    adv_ref[...] = jnp.sum(hw * ad_ref[...], axis=1, keepdims=True)


def _prep_mid(part, den, b, W, a_s, a_d):
    blk = 1000
    return pl.pallas_call(
        _prep_mid_body,
        grid=(NN // blk,),
        in_specs=[
            pl.BlockSpec((2, 2, blk, HD), lambda i: (0, 0, i, 0)),
            pl.BlockSpec((blk, 1), lambda i: (i, 0)),
            pl.BlockSpec((1, D), lambda i: (0, 0)),
            pl.BlockSpec((D, D), lambda i: (0, 0)),
            pl.BlockSpec((1, D), lambda i: (0, 0)),
            pl.BlockSpec((1, D), lambda i: (0, 0)),
        ],
        out_specs=[
            pl.BlockSpec((blk, D), lambda i: (i, 0)),
            pl.BlockSpec((blk, 1), lambda i: (i, 0)),
            pl.BlockSpec((blk, 1), lambda i: (i, 0)),
        ],
        out_shape=[
            jax.ShapeDtypeStruct((NN, D), jnp.float32),
            jax.ShapeDtypeStruct((NN, 1), jnp.float32),
            jax.ShapeDtypeStruct((NN, 1), jnp.float32),
        ],
    )(part, den, b, W, a_s, a_d)


def _combine_body(p_ref, d_ref, b_ref, h_ref):
    h_ref[...] = _halves_to_h(p_ref, d_ref, b_ref)


def _combine_h(part, den, b):
    blk = 1264
    return pl.pallas_call(
        _combine_body,
        grid=(NNA // blk,),
        in_specs=[
            pl.BlockSpec((2, 2, blk, HD), lambda i: (0, 0, i, 0)),
            pl.BlockSpec((blk, 1), lambda i: (i, 0)),
            pl.BlockSpec((1, D), lambda i: (0, 0)),
        ],
        out_specs=pl.BlockSpec((blk, D), lambda i: (i, 0)),
        out_shape=jax.ShapeDtypeStruct((NNA, D), jnp.float32),
    )(part, den, b)


def _final_body(hp_ref, nw_ref, w0_ref, b0_ref, wn_ref, bn_ref, w4a_ref,
                w4b_ref, b4_ref, o_ref):
    hp2 = jnp.maximum(
        jnp.dot(hp_ref[...], w0_ref[...], preferred_element_type=jnp.float32)
        + b0_ref[...], 0.0)
    nw2 = jnp.maximum(
        jnp.dot(nw_ref[...], wn_ref[...], preferred_element_type=jnp.float32)
        + bn_ref[...], 0.0)
    z = (jnp.dot(hp2, w4a_ref[...], preferred_element_type=jnp.float32)
         + jnp.dot(nw2, w4b_ref[...], preferred_element_type=jnp.float32)
         + b4_ref[...])
    o_ref[...] = 1.0 / (1.0 + jnp.exp(-z))


def _final_tc(hp, news, W0, b0, Wn, bn, W4a, W4b, b4):
    return pl.pallas_call(
        _final_body,
        out_shape=jax.ShapeDtypeStruct((NG, D), jnp.float32),
    )(hp, news, W0, b0, Wn, bn, W4a, W4b, b4)


def _gat_sc_body(hw_hbm, asv_hbm, adv_hbm, src_hbm, dst_hbm,
                 part_hbm, den_hbm,
                 u, efac, srcc, dstc, sbuf, zbuf, gbuf, tmp16,
                 acc_sh, den_sh, gmax_sh, gsem):
    c = lax.axis_index("c")
    s = lax.axis_index("s")
    zv = jnp.zeros((16,), jnp.float32)
    lanes = lax.iota(jnp.int32, 16)

    def zb(i, _):
        zbuf[pl.ds(i * 16, 16)] = zv
        return 0
    lax.fori_loop(0, 40, zb, 0)
    pltpu.sync_copy(zbuf, den_sh.at[pl.ds(s * 640, 640)])

    def zr(i, _):
        sbuf[i // 4, pl.ds((i % 4) * 16, 16)] = zv
        return 0
    lax.fori_loop(0, 512, zr, 0)

    def _zero_acc():
        for t in range(4):
            pltpu.sync_copy(sbuf,
                            acc_sh.at[pl.ds(s * 632 + t * 128, 128)])
        pltpu.sync_copy(sbuf.at[pl.ds(0, 120)],
                        acc_sh.at[pl.ds(s * 632 + 512, 120)])
    _zero_acc()

    pltpu.sync_copy(asv_hbm, u.at[pl.ds(0, NTAB)])
    pltpu.sync_copy(adv_hbm, u.at[pl.ds(NTAB, NTAB)])

    base = s * (ROWS_T * 128)
    neg = jnp.full((16,), -1e30, jnp.float32)

    lmax = neg
    for jc in range(2):
        pltpu.sync_copy(src_hbm.at[pl.ds(s * ROWS_T + jc * ROWS_W, ROWS_W)],
                        srcc)
        pltpu.sync_copy(dst_hbm.at[pl.ds(s * ROWS_T + jc * ROWS_W, ROWS_W)],
                        dstc)

        def p1(j2, lm, jc=jc):
            j = jc * ROWS_W + j2
            for g in range(8):
                sv = srcc[j2, pl.ds(g * 16, 16)]
                dv = dstc[j2, pl.ds(g * 16, 16)]
                av = plsc.load_gather(u, [_hi(sv), _lo(sv)])
                bv = plsc.load_gather(u, [_hi(dv) + NTAB, _lo(dv)])
                xv = av + bv
                ev = jnp.maximum(xv, 0.2 * xv)
                gidx = base + j * 128 + g * 16 + lanes
                ev = jnp.where(gidx < NE, ev, neg)
                efac[j, pl.ds(g * 16, 16)] = ev
                lm = jnp.maximum(lm, ev)
            return lm
        lmax = lax.fori_loop(0, ROWS_W, p1, lmax)

    tmp16[...] = lmax
    pltpu.sync_copy(tmp16, gmax_sh.at[s])
    plsc.subcore_barrier()
    pltpu.sync_copy(gmax_sh, gbuf)
    m = gbuf[0]
    for t in range(1, 16):
        m = jnp.maximum(m, gbuf[t])
    gv = jnp.full((16,), jnp.max(m))

    for jc in range(2):
        pltpu.sync_copy(dst_hbm.at[pl.ds(s * ROWS_T + jc * ROWS_W, ROWS_W)],
                        dstc)

        def p2(j2, _, jc=jc):
            j = jc * ROWS_W + j2
            for g in range(8):
                ev = efac[j, pl.ds(g * 16, 16)]
                efac[j, pl.ds(g * 16, 16)] = jnp.exp(ev - gv)
            pltpu.sync_copy(efac.at[j], den_sh.at[dstc.at[j2]], add=True)
            return 0
        lax.fori_loop(0, ROWS_W, p2, 0)

    pltpu.sync_copy(src_hbm.at[pl.ds(s * ROWS_T + c * ROWS_W, ROWS_W)], srcc)
    pltpu.sync_copy(dst_hbm.at[pl.ds(s * ROWS_T + c * ROWS_W, ROWS_W)], dstc)
    r0 = c * ROWS_W
    for h in range(2):
        if h == 1:
            plsc.subcore_barrier()

        def rp(j, _, h=h):
            pltpu.async_copy(hw_hbm.at[srcc.at[j]], u.at[pl.ds(0, 128)],
                             gsem).wait()

            def sc_row(r, _):
                al = plsc.load_gather(
                    efac, [jnp.broadcast_to(r0 + j, (16,)).astype(jnp.int32),
                           jnp.broadcast_to(r, (16,)).astype(jnp.int32)])
                for g in range(4):
                    sbuf[r, pl.ds(g * 16, 16)] = (
                        u[r, pl.ds(h * HD + g * 16, 16)] * al)
                return 0
            lax.fori_loop(0, 128, sc_row, 0)
            pltpu.sync_copy(sbuf, acc_sh.at[dstc.at[j]], add=True)
            return 0
        lax.fori_loop(0, ROWS_W, rp, 0)

        plsc.subcore_barrier()
        pltpu.sync_copy(acc_sh.at[pl.ds(s * 632, 632)],
                        part_hbm.at[c, h, pl.ds(s * 632, 632)])
        if h == 0:
            def zr2(i, _):
                sbuf[i // 4, pl.ds((i % 4) * 16, 16)] = zv
                return 0
            lax.fori_loop(0, 512, zr2, 0)
            _zero_acc()

    @pl.when(c == 0)
    def _():
        pltpu.sync_copy(den_sh.at[pl.ds(s * 640, 640)],
                        den_hbm.at[pl.ds(s * 640, 640)])


def _gat_sc(hw, asv, adv, srcp, dstp):
    f = functools.partial(
        pl.kernel,
        out_type=(jax.ShapeDtypeStruct((NC, 2, NNA, HD), jnp.float32),
                  jax.ShapeDtypeStruct((NNP,), jnp.float32)),
        mesh=_mesh(),
        scratch_types=[
            pltpu.VMEM((2 * NTAB, 128), jnp.float32),
            pltpu.VMEM((ROWS_T, 128), jnp.float32),
            pltpu.VMEM((ROWS_W, 128), jnp.int32),
            pltpu.VMEM((ROWS_W, 128), jnp.int32),
            pltpu.VMEM((128, HD), jnp.float32),
            pltpu.VMEM((640,), jnp.float32),
            pltpu.VMEM((16, 16), jnp.float32),
            pltpu.VMEM((16,), jnp.float32),
            pltpu.VMEM_SHARED((NNA, HD), jnp.float32),
            pltpu.VMEM_SHARED((NNP,), jnp.float32),
            pltpu.VMEM_SHARED((16, 16), jnp.float32),
            pltpu.SemaphoreType.DMA,
        ],
        compiler_params=pltpu.CompilerParams(needs_layout_passes=False, use_tc_tiling_on_sc=False),
    )(_gat_sc_body)
    return f(hw, asv, adv, srcp, dstp)


def _pool_sc_body(h_hbm, bat_hbm, x_hbm, hp_hbm, news_hbm,
                  batv, ones, histv, startv, cbuf, hpbuf, ridx, nbufg, zbuf,
                  hist_sh, gsem):
    c = lax.axis_index("c")
    s = lax.axis_index("s")
    wid = s * NC + c
    zv = jnp.zeros((16,), jnp.float32)
    ov = jnp.ones((16,), jnp.float32)
    lanes = lax.iota(jnp.int32, 16)

    for i in range(NBIN // 16):
        zbuf[pl.ds(i * 16, 16)] = zv

    @pl.when(s == 0)
    def _():
        pltpu.sync_copy(zbuf, hist_sh)
    for g in range(8):
        ones[0, pl.ds(g * 16, 16)] = ov
    pltpu.sync_copy(bat_hbm, batv)
    plsc.subcore_barrier()

    for j in range(5):
        pltpu.sync_copy(ones.at[0], hist_sh.at[batv.at[s * 5 + j]], add=True)
    plsc.subcore_barrier()
    for kb in range(NBIN // 16):
        pltpu.sync_copy(hist_sh.at[pl.ds(kb * 16, 16)], histv.at[kb])

    run = jnp.zeros((), jnp.float32)
    for kb in range(NBIN // 16):
        v = histv[kb]
        inc = plsc.cumsum(v)
        startv[kb] = inc - v + run
        run = run + jnp.sum(v)

    def scalar_at(ref, g):
        vec = plsc.load_gather(
            ref, [jnp.broadcast_to(lax.div(g, 16), (16,)),
                  jnp.broadcast_to(lax.rem(g, 16), (16,))])
        return jnp.max(vec)

    roots = []
    for gi in range(2):
        g = wid * 2 + gi
        g_start = scalar_at(startv, g).astype(jnp.int32)
        g_cnt = scalar_at(histv, g).astype(jnp.int32)
        roots.append(g_start)
        g0 = pl.multiple_of((g_start // 8) * 8, 8)

        def chunk(kk, acc, g0=g0, g_start=g_start, g_cnt=g_cnt):
            off = pl.multiple_of(g0 + kk * 32, 8)
            pltpu.sync_copy(h_hbm.at[pl.ds(off, 32)], cbuf)

            def rbody(r, acc2):
                row = off + r
                valid = jnp.logical_and(row >= g_start,
                                        row < g_start + g_cnt)
                out = []
                for gg in range(8):
                    v = cbuf[r, pl.ds(gg * 16, 16)]
                    out.append(jnp.where(valid, jnp.maximum(acc2[gg], v),
                                         acc2[gg]))
                return tuple(out)
            return lax.fori_loop(0, 32, rbody, acc)

        kmax = lax.div(g_start + g_cnt - g0 + 31, 32)
        acc0 = tuple(zv for _ in range(8))
        acc = lax.fori_loop(0, kmax, chunk, acc0)
        for gg in range(8):
            hpbuf[gi, pl.ds(gg * 16, 16)] = acc[gg]

    ridx[...] = jnp.where(lanes == 0, roots[0], roots[1])
    pltpu.async_copy(x_hbm.at[ridx], nbufg, gsem).wait()

    pltpu.sync_copy(hpbuf, hp_hbm.at[wid])
    pltpu.sync_copy(nbufg.at[pl.ds(0, 2)], news_hbm.at[wid])


def _pool_sc(h3, batp, x):
    f = functools.partial(
        pl.kernel,
        out_type=(jax.ShapeDtypeStruct((NS * NC, 2, D), jnp.float32),
                  jax.ShapeDtypeStruct((NS * NC, 2, D), jnp.float32)),
        mesh=_mesh(),
        scratch_types=[
            pltpu.VMEM((NNP // 128, 128), jnp.int32),
            pltpu.VMEM((1, 128), jnp.float32),
            pltpu.VMEM((5, 16), jnp.float32),
            pltpu.VMEM((5, 16), jnp.float32),
            pltpu.VMEM((32, D), jnp.float32),
            pltpu.VMEM((2, D), jnp.float32),
            pltpu.VMEM((16,), jnp.int32),
            pltpu.VMEM((16, D), jnp.float32),
            pltpu.VMEM((NBIN,), jnp.float32),
            pltpu.VMEM_SHARED((NBIN,), jnp.float32),
            pltpu.SemaphoreType.DMA,
        ],
        compiler_params=pltpu.CompilerParams(needs_layout_passes=False, use_tc_tiling_on_sc=False),
    )(_pool_sc_body)
    return f(h3, batp, x)


def _pad_tab(v):
    return jnp.pad(v.reshape(NN), (0, NNP - NN)).reshape(NTAB, 128)


def kernel(x, edge_index, batch, W1, as1, ad1, b1, W2, as2, ad2, b2,
           W3, as3, ad3, b3, Wn, bn, W0, b0, W4, b4):
    src, dst = edge_index[0], edge_index[1]
    pad = NEP - NE
    srcp = jnp.concatenate([src, jnp.zeros((pad,), jnp.int32)]).reshape(NEP // 128, 128)
    dstp = jnp.concatenate([dst, jnp.zeros((pad,), jnp.int32)]).reshape(NEP // 128, 128)
    batp = jnp.concatenate(
        [batch, jnp.full((NNP - NN,), NG, jnp.int32)]).reshape(NNP // 128, 128)

    hw, asv, adv = _prep_first(x, W1, as1.reshape(1, D), ad1.reshape(1, D))
    part, den = _gat_sc(hw, _pad_tab(asv), _pad_tab(adv), srcp, dstp)
    hw, asv, adv = _prep_mid(part, den.reshape(NNP, 1),
                             b1.reshape(1, D), W2,
                             as2.reshape(1, D), ad2.reshape(1, D))
    part, den = _gat_sc(hw, _pad_tab(asv), _pad_tab(adv), srcp, dstp)
    hw, asv, adv = _prep_mid(part, den.reshape(NNP, 1),
                             b2.reshape(1, D), W3,
                             as3.reshape(1, D), ad3.reshape(1, D))
    part, den = _gat_sc(hw, _pad_tab(asv), _pad_tab(adv), srcp, dstp)

    h3 = _combine_h(part, den.reshape(NNP, 1), b3.reshape(1, D))[:NN]
    hp = jax.ops.segment_max(h3, batch, num_segments=NG)
    hp = jax.nn.relu(hp @ W0 + b0)
    diff = batch[1:] - batch[:-1]
    root = jnp.nonzero(diff, size=NG - 1)[0]
    root = jnp.concatenate([jnp.zeros((1,), dtype=root.dtype), root + 1])
    news = jax.nn.relu(x[root] @ Wn + bn)
    out = jnp.concatenate([hp, news], axis=-1) @ W4 + b4
    return jax.nn.sigmoid(out)
